# baseline (speedup 1.0000x reference)
import functools

import jax
import jax.numpy as jnp
from jax.experimental import pallas as pl

N_I = 256
N_P = 8192
N_C = 9
P_BLK = 8192


def _rot_kernel(w_ref, x_ref, o_ref):
    x = x_ref[0]
    w = w_ref[0]
    acc = x[:, 0:1] * w[0:1, :]
    acc = acc + x[:, 1:2] * w[1:2, :]
    acc = acc + x[:, 2:3] * w[2:3, :]
    lane = jax.lax.broadcasted_iota(jnp.int32, (P_BLK, N_C), 1)
    o_ref[0] = jnp.where(lane < 3, acc, x)


@functools.partial(jax.jit, static_argnames=("interpret",))
def kernel(points_colored_instance, rot_mats, interpret=False):
    w = jnp.swapaxes(rot_mats, 1, 2)
    w = jnp.pad(w, ((0, 0), (0, 0), (0, N_C - 3)))
    out = pl.pallas_call(
        _rot_kernel,
        grid=(N_I, N_P // P_BLK),
        in_specs=[
            pl.BlockSpec((1, 3, N_C), lambda i, j: (i, 0, 0)),
            pl.BlockSpec((1, P_BLK, N_C), lambda i, j: (i, j, 0)),
        ],
        out_specs=pl.BlockSpec((1, P_BLK, N_C), lambda i, j: (i, j, 0)),
        out_shape=jax.ShapeDtypeStruct((N_I, N_P, N_C), jnp.float32),
        interpret=interpret,
    )(w, points_colored_instance)
    return out


# speedup vs baseline: 35.9452x; 35.9452x over previous
import functools

import jax
import jax.numpy as jnp
from jax.experimental import pallas as pl

N_I = 256
N_P = 8192
N_C = 9
I_BLK = 8
P_BLK = 8192


def _rot_plane_kernel(w_ref, x_ref, o_ref):
    w = w_ref[...]
    for d in range(3):
        acc = x_ref[0] * w[:, 3 * d : 3 * d + 1]
        acc += x_ref[1] * w[:, 3 * d + 1 : 3 * d + 2]
        acc += x_ref[2] * w[:, 3 * d + 2 : 3 * d + 3]
        o_ref[d] = acc
    for c in range(3, N_C):
        o_ref[c] = x_ref[c]


@functools.partial(jax.jit, static_argnames=("interpret",))
def kernel(points_colored_instance, rot_mats, interpret=False):
    xt = jnp.transpose(points_colored_instance, (2, 0, 1))
    w = rot_mats.reshape(N_I, 9)
    out = pl.pallas_call(
        _rot_plane_kernel,
        grid=(N_I // I_BLK, N_P // P_BLK),
        in_specs=[
            pl.BlockSpec((I_BLK, 9), lambda i, j: (i, 0)),
            pl.BlockSpec((N_C, I_BLK, P_BLK), lambda i, j: (0, i, j)),
        ],
        out_specs=pl.BlockSpec((N_C, I_BLK, P_BLK), lambda i, j: (0, i, j)),
        out_shape=jax.ShapeDtypeStruct((N_C, N_I, N_P), jnp.float32),
        interpret=interpret,
    )(w, xt)
    return jnp.transpose(out, (1, 2, 0))


# speedup vs baseline: 38.9329x; 1.0831x over previous
import functools

import jax
import jax.numpy as jnp
from jax.experimental import pallas as pl

N_I = 256
N_P = 8192
N_C = 9
I_BLK = 16
P_BLK = 8192


def _rot_plane_kernel(w_ref, x_ref, o_ref):
    w = w_ref[...]
    for d in range(3):
        acc = x_ref[0] * w[:, 3 * d : 3 * d + 1]
        acc += x_ref[1] * w[:, 3 * d + 1 : 3 * d + 2]
        acc += x_ref[2] * w[:, 3 * d + 2 : 3 * d + 3]
        o_ref[d] = acc
    for c in range(3, N_C):
        o_ref[c] = x_ref[c]


@functools.partial(jax.jit, static_argnames=("interpret",))
def kernel(points_colored_instance, rot_mats, interpret=False):
    xt = jnp.transpose(points_colored_instance, (2, 0, 1))
    w = rot_mats.reshape(N_I, 9)
    out = pl.pallas_call(
        _rot_plane_kernel,
        grid=(N_I // I_BLK, N_P // P_BLK),
        in_specs=[
            pl.BlockSpec((I_BLK, 9), lambda i, j: (i, 0)),
            pl.BlockSpec((N_C, I_BLK, P_BLK), lambda i, j: (0, i, j)),
        ],
        out_specs=pl.BlockSpec((N_C, I_BLK, P_BLK), lambda i, j: (0, i, j)),
        out_shape=jax.ShapeDtypeStruct((N_C, N_I, N_P), jnp.float32),
        interpret=interpret,
    )(w, xt)
    return jnp.transpose(out, (1, 2, 0))


# speedup vs baseline: 39.8040x; 1.0224x over previous
import functools

import jax
import jax.numpy as jnp
from jax.experimental import pallas as pl

N_I = 256
N_P = 8192
N_C = 9
I_BLK = 32
P_BLK = 8192


def _rot_plane_kernel(w_ref, x_ref, o_ref):
    w = w_ref[...]
    for d in range(3):
        acc = x_ref[0] * w[:, 3 * d : 3 * d + 1]
        acc += x_ref[1] * w[:, 3 * d + 1 : 3 * d + 2]
        acc += x_ref[2] * w[:, 3 * d + 2 : 3 * d + 3]
        o_ref[d] = acc
    for c in range(3, N_C):
        o_ref[c] = x_ref[c]


@functools.partial(jax.jit, static_argnames=("interpret",))
def kernel(points_colored_instance, rot_mats, interpret=False):
    xt = jnp.transpose(points_colored_instance, (2, 0, 1))
    w = rot_mats.reshape(N_I, 9)
    out = pl.pallas_call(
        _rot_plane_kernel,
        grid=(N_I // I_BLK, N_P // P_BLK),
        in_specs=[
            pl.BlockSpec((I_BLK, 9), lambda i, j: (i, 0)),
            pl.BlockSpec((N_C, I_BLK, P_BLK), lambda i, j: (0, i, j)),
        ],
        out_specs=pl.BlockSpec((N_C, I_BLK, P_BLK), lambda i, j: (0, i, j)),
        out_shape=jax.ShapeDtypeStruct((N_C, N_I, N_P), jnp.float32),
        interpret=interpret,
    )(w, xt)
    return jnp.transpose(out, (1, 2, 0))
